# Initial kernel scaffold; baseline (speedup 1.0000x reference)
import jax
import jax.numpy as jnp
from jax.experimental import pallas as pl


def kernel(x, prev_q, codes):
    raise NotImplementedError("write your pallas kernel here")


# speedup vs baseline: 1.4067x; 1.4067x over previous
import functools

import jax
import jax.numpy as jnp
from jax import lax
from jax.experimental import pallas as pl

B = 4096
D = 32
K = 8192
BETA = 0.25
EPS = 1e-6


def _vq_body(nt, x_ref, pq_ref, codes_ref, quant_ref, idx_ref, loss_ref):
    x = x_ref[...]
    pq = pq_ref[...]
    codes = codes_ref[...]

    nrm = jnp.sqrt(jnp.sum(pq * pq, axis=1, keepdims=True))
    u = pq / jnp.maximum(nrm, EPS)
    inv = jnp.float32(1.0) / jnp.sqrt(jnp.float32(D))

    A = u[:, :, None] * inv - inv * u[:, None, :]
    A2 = lax.dot_general(A, A, (((2,), (1,)), ((0,), (0,))),
                         preferred_element_type=jnp.float32)
    dotuv = jnp.sum(u * inv, axis=1)[:, None, None]
    ii = lax.broadcasted_iota(jnp.int32, (D, D), 0)
    jj = lax.broadcasted_iota(jnp.int32, (D, D), 1)
    eye = (ii == jj).astype(jnp.float32)
    Rm = eye[None, :, :] + A + A2 / (1.0 + dotuv + EPS)

    xc = jnp.sum(Rm * x[:, :, None], axis=1)

    dot = lax.dot_general(xc, codes, (((1,), (1,)), ((), ())),
                          preferred_element_type=jnp.float32)
    x_sq = jnp.sum(xc * xc, axis=1, keepdims=True)
    c_sq = jnp.sum(codes * codes, axis=1)[None, :]
    dist = x_sq - 2.0 * dot + c_sq
    minval = jnp.min(dist, axis=1, keepdims=True)
    kiota = lax.broadcasted_iota(jnp.int32, dist.shape, 1)
    idx = jnp.min(jnp.where(dist == minval, kiota, K), axis=1)
    idx_ref[...] = idx

    onehot = (kiota == idx[:, None]).astype(jnp.float32)
    qc = lax.dot_general(onehot, codes, (((1,), (0,)), ((), ())),
                         preferred_element_type=jnp.float32)

    quant_ref[...] = jnp.sum(Rm * qc[:, None, :], axis=2)

    part = jnp.sum((x - qc) ** 2).reshape(1, 1)

    @pl.when(pl.program_id(0) == 0)
    def _init():
        loss_ref[...] = jnp.zeros((1, 1), jnp.float32)

    loss_ref[...] += part

    @pl.when(pl.program_id(0) == nt - 1)
    def _finish():
        loss_ref[...] = loss_ref[...] * ((1.0 + BETA) / B)


@functools.partial(jax.jit, static_argnames=("bt",))
def _run(x, prev_q, codes2d, bt=256):
    nt = B // bt
    quant, idx, loss = pl.pallas_call(
        functools.partial(_vq_body, nt),
        grid=(nt,),
        in_specs=[
            pl.BlockSpec((bt, D), lambda i: (i, 0)),
            pl.BlockSpec((bt, D), lambda i: (i, 0)),
            pl.BlockSpec((K, D), lambda i: (0, 0)),
        ],
        out_specs=[
            pl.BlockSpec((bt, D), lambda i: (i, 0)),
            pl.BlockSpec((bt,), lambda i: (i,)),
            pl.BlockSpec((1, 1), lambda i: (0, 0)),
        ],
        out_shape=[
            jax.ShapeDtypeStruct((B, D), jnp.float32),
            jax.ShapeDtypeStruct((B,), jnp.int32),
            jax.ShapeDtypeStruct((1, 1), jnp.float32),
        ],
    )(x, prev_q, codes2d)
    return quant, idx, loss[0, 0]


def kernel(x, prev_q, codes):
    return _run(x, prev_q, codes[0])


# speedup vs baseline: 1.4107x; 1.0029x over previous
import functools

import jax
import jax.numpy as jnp
from jax import lax
from jax.experimental import pallas as pl
from jax.experimental.pallas import tpu as pltpu

B = 4096
D = 32
K = 8192
BETA = 0.25
EPS = 1e-6


def _vq_body(nt, x_ref, pq_ref, codes_ref, eye_ref,
             quant_ref, idx_ref, loss_ref, csq_ref):
    x = x_ref[...]
    pq = pq_ref[...]
    codes = codes_ref[...]

    @pl.when(pl.program_id(0) == 0)
    def _csq():
        csq_ref[...] = jnp.sum(codes * codes, axis=1)[None, :]

    nrm = jnp.sqrt(jnp.sum(pq * pq, axis=1, keepdims=True))
    u = pq / jnp.maximum(nrm, EPS)
    inv = jnp.float32(1.0) / jnp.sqrt(jnp.float32(D))
    w = u * inv

    A = w[:, :, None] - w[:, None, :]

    A2 = lax.dot_general(A, A, (((2,), (1,)), ((0,), (0,))),
                         preferred_element_type=jnp.float32)
    dotuv = jnp.sum(w, axis=1)[:, None, None]
    eye = eye_ref[...]
    Rm = eye[None, :, :] + A + A2 / (1.0 + dotuv + EPS)

    xc = jnp.sum(Rm * x[:, :, None], axis=1)

    dot = lax.dot_general(xc, codes, (((1,), (1,)), ((), ())),
                          preferred_element_type=jnp.float32)
    x_sq = jnp.sum(xc * xc, axis=1, keepdims=True)
    dist = x_sq - 2.0 * dot + csq_ref[...]
    kiota = lax.broadcasted_iota(jnp.int32, dist.shape, 1)
    idx = jnp.argmin(dist, axis=1).astype(jnp.int32)
    idx_ref[...] = idx

    onehot = (kiota == idx[:, None]).astype(jnp.float32)
    qc = lax.dot_general(onehot, codes, (((1,), (0,)), ((), ())),
                         preferred_element_type=jnp.float32)

    quant_ref[...] = lax.dot_general(Rm, qc[:, :, None], (((2,), (1,)), ((0,), (0,))),
                                     preferred_element_type=jnp.float32)[:, :, 0]

    part = jnp.sum((x - qc) ** 2).reshape(1, 1)

    @pl.when(pl.program_id(0) == 0)
    def _init():
        loss_ref[...] = jnp.zeros((1, 1), jnp.float32)

    loss_ref[...] += part

    @pl.when(pl.program_id(0) == nt - 1)
    def _finish():
        loss_ref[...] = loss_ref[...] * ((1.0 + BETA) / B)


@functools.partial(jax.jit, static_argnames=("bt",))
def _run(x, prev_q, codes2d, bt=256):
    nt = B // bt
    quant, idx, loss = pl.pallas_call(
        functools.partial(_vq_body, nt),
        grid=(nt,),
        in_specs=[
            pl.BlockSpec((bt, D), lambda i: (i, 0)),
            pl.BlockSpec((bt, D), lambda i: (i, 0)),
            pl.BlockSpec((K, D), lambda i: (0, 0)),
            pl.BlockSpec((D, D), lambda i: (0, 0)),
        ],
        out_specs=[
            pl.BlockSpec((bt, D), lambda i: (i, 0)),
            pl.BlockSpec((bt,), lambda i: (i,)),
            pl.BlockSpec((1, 1), lambda i: (0, 0)),
        ],
        out_shape=[
            jax.ShapeDtypeStruct((B, D), jnp.float32),
            jax.ShapeDtypeStruct((B,), jnp.int32),
            jax.ShapeDtypeStruct((1, 1), jnp.float32),
        ],
        scratch_shapes=[pltpu.VMEM((1, K), jnp.float32)],
    )(x, prev_q, codes2d, jnp.eye(D, dtype=jnp.float32))
    return quant, idx, loss[0, 0]


def kernel(x, prev_q, codes):
    return _run(x, prev_q, codes[0])


# speedup vs baseline: 1.4929x; 1.0583x over previous
import functools

import jax
import jax.numpy as jnp
from jax import lax
from jax.experimental import pallas as pl
from jax.experimental.pallas import tpu as pltpu

B = 4096
D = 32
K = 8192
BETA = 0.25
EPS = 1e-6


def _vq_body(nt, x_ref, pq_ref, codes_ref, eye_ref,
             quant_ref, idx_ref, loss_ref, csq_ref):
    x = x_ref[...]
    pq = pq_ref[...]
    codes = codes_ref[...]

    @pl.when(pl.program_id(0) == 0)
    def _csq():
        csq_ref[...] = jnp.sum(codes * codes, axis=1)[None, :]

    nrm = jnp.sqrt(jnp.sum(pq * pq, axis=1, keepdims=True))
    u = pq / jnp.maximum(nrm, EPS)
    inv = jnp.float32(1.0) / jnp.sqrt(jnp.float32(D))
    w = u * inv

    A = w[:, :, None] - w[:, None, :]

    A2 = lax.dot_general(A, A, (((2,), (1,)), ((0,), (0,))),
                         preferred_element_type=jnp.float32)
    dotuv = jnp.sum(w, axis=1)[:, None, None]
    eye = eye_ref[...]
    Rm = eye[None, :, :] + A + A2 / (1.0 + dotuv + EPS)

    xc = jnp.sum(Rm * x[:, :, None], axis=1)

    dot = lax.dot_general(xc, codes, (((1,), (1,)), ((), ())),
                          preferred_element_type=jnp.float32)
    x_sq = jnp.sum(xc * xc, axis=1, keepdims=True)
    dist = x_sq - 2.0 * dot + csq_ref[...]
    kiota = lax.broadcasted_iota(jnp.int32, dist.shape, 1)
    idx = jnp.argmin(dist, axis=1).astype(jnp.int32)
    idx_ref[...] = idx

    onehot = (kiota == idx[:, None]).astype(jnp.float32)
    qc = lax.dot_general(onehot, codes, (((1,), (0,)), ((), ())),
                         preferred_element_type=jnp.float32)

    quant_ref[...] = lax.dot_general(Rm, qc[:, :, None], (((2,), (1,)), ((0,), (0,))),
                                     preferred_element_type=jnp.float32)[:, :, 0]

    part = jnp.sum((x - qc) ** 2).reshape(1, 1)

    @pl.when(pl.program_id(0) == 0)
    def _init():
        loss_ref[...] = jnp.zeros((1, 1), jnp.float32)

    loss_ref[...] += part

    @pl.when(pl.program_id(0) == nt - 1)
    def _finish():
        loss_ref[...] = loss_ref[...] * ((1.0 + BETA) / B)


@functools.partial(jax.jit, static_argnames=("bt",))
def _run(x, prev_q, codes2d, bt=512):
    nt = B // bt
    quant, idx, loss = pl.pallas_call(
        functools.partial(_vq_body, nt),
        grid=(nt,),
        in_specs=[
            pl.BlockSpec((bt, D), lambda i: (i, 0)),
            pl.BlockSpec((bt, D), lambda i: (i, 0)),
            pl.BlockSpec((K, D), lambda i: (0, 0)),
            pl.BlockSpec((D, D), lambda i: (0, 0)),
        ],
        out_specs=[
            pl.BlockSpec((bt, D), lambda i: (i, 0)),
            pl.BlockSpec((bt,), lambda i: (i,)),
            pl.BlockSpec((1, 1), lambda i: (0, 0)),
        ],
        out_shape=[
            jax.ShapeDtypeStruct((B, D), jnp.float32),
            jax.ShapeDtypeStruct((B,), jnp.int32),
            jax.ShapeDtypeStruct((1, 1), jnp.float32),
        ],
        scratch_shapes=[pltpu.VMEM((1, K), jnp.float32)],
    )(x, prev_q, codes2d, jnp.eye(D, dtype=jnp.float32))
    return quant, idx, loss[0, 0]


def kernel(x, prev_q, codes):
    return _run(x, prev_q, codes[0])
